# baseline (speedup 1.0000x reference)
import jax
import jax.numpy as jnp
from jax.experimental import pallas as pl
from jax.experimental.pallas import tpu as pltpu


def _dma_body(x_ref, o_ref, sem):
    C = 3
    for c in range(C):
        pltpu.make_async_copy(x_ref.at[:, (C - 1) - c], o_ref.at[:, c], sem.at[c]).start()
    for c in range(C):
        pltpu.make_async_copy(x_ref.at[:, (C - 1) - c], o_ref.at[:, c], sem.at[c]).wait()


def kernel(x):
    return pl.pallas_call(
        _dma_body,
        in_specs=[pl.BlockSpec(memory_space=pl.ANY)],
        out_specs=pl.BlockSpec(memory_space=pl.ANY),
        out_shape=jax.ShapeDtypeStruct(x.shape, x.dtype),
        scratch_shapes=[pltpu.SemaphoreType.DMA((3,))],
    )(x)


# speedup vs baseline: 43.9805x; 43.9805x over previous
import jax
import jax.numpy as jnp
from jax.experimental import pallas as pl


def _copy_body(x_ref, o_ref):
    o_ref[...] = x_ref[...]


def kernel(x):
    B, C, H, W = x.shape
    BB = 2
    return pl.pallas_call(
        _copy_body,
        grid=(B // BB, C),
        in_specs=[pl.BlockSpec((BB, 1, H, W), lambda b, c: (b, (C - 1) - c, 0, 0))],
        out_specs=pl.BlockSpec((BB, 1, H, W), lambda b, c: (b, c, 0, 0)),
        out_shape=jax.ShapeDtypeStruct(x.shape, x.dtype),
    )(x)


# speedup vs baseline: 47.5724x; 1.0817x over previous
import jax
import jax.numpy as jnp
from jax.experimental import pallas as pl


def _copy_body(x_ref, o_ref):
    o_ref[...] = x_ref[...]


def kernel(x):
    B, C, H, W = x.shape
    BB = 4
    return pl.pallas_call(
        _copy_body,
        grid=(B // BB, C),
        in_specs=[pl.BlockSpec((BB, 1, H, W), lambda b, c: (b, (C - 1) - c, 0, 0))],
        out_specs=pl.BlockSpec((BB, 1, H, W), lambda b, c: (b, c, 0, 0)),
        out_shape=jax.ShapeDtypeStruct(x.shape, x.dtype),
    )(x)


# speedup vs baseline: 48.5216x; 1.0200x over previous
import jax
import jax.numpy as jnp
from jax.experimental import pallas as pl


def _copy_body(x_ref, o_ref):
    o_ref[...] = x_ref[...]


def kernel(x):
    B, C, H, W = x.shape
    BB = 8
    return pl.pallas_call(
        _copy_body,
        grid=(B // BB, C),
        in_specs=[pl.BlockSpec((BB, 1, H, W), lambda b, c: (b, (C - 1) - c, 0, 0))],
        out_specs=pl.BlockSpec((BB, 1, H, W), lambda b, c: (b, c, 0, 0)),
        out_shape=jax.ShapeDtypeStruct(x.shape, x.dtype),
    )(x)
